# baseline (speedup 1.0000x reference)
import functools

import jax
import jax.numpy as jnp
from jax import lax
from jax.experimental import pallas as pl
from jax.experimental.pallas import tpu as pltpu
from jax.experimental.pallas import tpu_sc as plsc

_N = 10000
_E = 320000
_D0 = 128
_H = 150
_HP = 160
_NC = 2
_NS = 16
_NW = _NC * _NS
_EPT = _E // _NW
_C = 80
_NCH = _EPT // _C
_RPTP = 640
_NP = _RPTP * _NS
_CS = 80
_KSEG = _RPTP // _CS
_R = 1000
_G = _N // _R


def _sc_mesh():
    return plsc.VectorSubcoreMesh(core_axis_name="c", subcore_axis_name="s")


@functools.partial(
    pl.kernel,
    mesh=_sc_mesh(),
    out_type=jax.ShapeDtypeStruct((_NC * 2 * _NP,), jnp.float32),
    compiler_params=pltpu.CompilerParams(use_tc_tiling_on_sc=False),
    scratch_types=[
        pltpu.VMEM((_NCH, 2, _C), jnp.int32),
        pltpu.VMEM((_C,), jnp.float32),
        pltpu.VMEM((_RPTP,), jnp.float32),
        pltpu.VMEM_SHARED((_NP,), jnp.float32),
        pltpu.VMEM_SHARED((_NP,), jnp.float32),
        pltpu.SemaphoreType.DMA,
        pltpu.SemaphoreType.DMA,
    ],
)
def _deg_kernel(idx_hbm, out_hbm, ixall, ones, stage, degs, degd, ds0, ds1):
    c = lax.axis_index("c")
    s = lax.axis_index("s")
    ch0 = (s * _NC + c) * _NCH
    pltpu.sync_copy(idx_hbm.at[pl.ds(ch0, _NCH)], ixall)
    for j in range(_C // 16):
        ones[pl.ds(j * 16, 16)] = jnp.ones((16,), jnp.float32)

    def zfill(j, carry):
        stage[pl.ds(j * 16, 16)] = jnp.zeros((16,), jnp.float32)
        return carry

    lax.fori_loop(0, _RPTP // 16, zfill, 0)
    r0 = s * _RPTP
    pltpu.sync_copy(stage, degs.at[pl.ds(r0, _RPTP)])
    pltpu.sync_copy(stage, degd.at[pl.ds(r0, _RPTP)])
    plsc.subcore_barrier()
    one_c = ones

    def body(i, carry):
        pltpu.sync_copy(one_c, degs.at[ixall.at[i, 0]], add=True)
        pltpu.sync_copy(one_c, degd.at[ixall.at[i, 1]], add=True)
        return carry

    lax.fori_loop(0, _NCH, body, 0)
    plsc.subcore_barrier()
    pltpu.sync_copy(degs.at[pl.ds(r0, _RPTP)], stage)
    pltpu.sync_copy(stage, out_hbm.at[pl.ds(c * 2 * _NP + r0, _RPTP)])
    pltpu.sync_copy(degd.at[pl.ds(r0, _RPTP)], stage)
    pltpu.sync_copy(stage, out_hbm.at[pl.ds((c * 2 + 1) * _NP + r0, _RPTP)])


def _make_agg(D):
    @functools.partial(
        pl.kernel,
        mesh=_sc_mesh(),
        out_type=jax.ShapeDtypeStruct((_NC, _NP, D), jnp.float32),
        compiler_params=pltpu.CompilerParams(use_tc_tiling_on_sc=False),
        scratch_types=[
            pltpu.VMEM((2, _C), jnp.int32),
            pltpu.VMEM((2, _C), jnp.int32),
            pltpu.VMEM((2, _C), jnp.int32),
            pltpu.VMEM((_C, D), jnp.float32),
            pltpu.VMEM((_C, D), jnp.float32),
            pltpu.VMEM_SHARED((_NP, D), jnp.float32),
            pltpu.SemaphoreType.DMA,
            pltpu.SemaphoreType.DMA,
            pltpu.SemaphoreType.DMA,
            pltpu.SemaphoreType.DMA,
            pltpu.SemaphoreType.DMA,
            pltpu.SemaphoreType.DMA,
            pltpu.SemaphoreType.DMA,
        ],
    )
    def agg_kernel(h_hbm, idx_hbm, out_hbm, ix0, ix1, ix2, rows_a, rows_b,
                   acc, gs0, gs1, ss0, ss1, is0, is1, is2):
        c = lax.axis_index("c")
        s = lax.axis_index("s")
        r0 = s * _RPTP
        ch0 = (s * _NC + c) * _NCH
        ix = [ix0, ix1, ix2]
        rows = [rows_a, rows_b]
        gs = [gs0, gs1]
        ss = [ss0, ss1]
        isem = [is0, is1, is2]

        def zfill(r, carry):
            for j in range(D // 16):
                rows_a[r, pl.ds(j * 16, 16)] = jnp.zeros((16,), jnp.float32)
            return carry

        lax.fori_loop(0, _CS, zfill, 0)
        seg = rows_a.at[: _CS]

        def zcopy(k, carry):
            pltpu.sync_copy(seg, acc.at[pl.ds(r0 + k * _CS, _CS)])
            return carry

        lax.fori_loop(0, _KSEG, zcopy, 0)
        plsc.subcore_barrier()

        def step(i, t, kind):
            x = t % 2
            xn = (t + 1) % 2
            p = t % 3
            pn = (t + 1) % 3
            pnn = (t + 2) % 3
            pltpu.make_async_copy(h_hbm.at[ix[p].at[0]], rows[x], gs[x]).wait()
            if kind == "final":
                pltpu.sync_copy(rows[x], acc.at[ix[p].at[1]], add=True)
                return
            pltpu.make_async_copy(idx_hbm.at[i + 1], ix[pn], isem[pn]).wait()
            pltpu.async_copy(h_hbm.at[ix[pn].at[0]], rows[xn], gs[xn])
            pltpu.sync_copy(rows[x], acc.at[ix[p].at[1]], add=True)
            if kind == "full":
                pltpu.async_copy(idx_hbm.at[i + 2], ix[pnn], isem[pnn])

        pltpu.sync_copy(idx_hbm.at[ch0], ix0)
        pltpu.async_copy(h_hbm.at[ix0.at[0]], rows_a, gs0)
        pltpu.async_copy(idx_hbm.at[ch0 + 1], ix1, is1)

        nmain = (_NCH - 4) // 6

        def body(k, carry):
            i0 = ch0 + 6 * k
            for t in range(6):
                step(i0 + t, t, "full")
            return carry

        lax.fori_loop(0, nmain, body, 0)
        for t in range(nmain * 6, _NCH):
            if t < _NCH - 2:
                kind = "full"
            elif t == _NCH - 2:
                kind = "noidx"
            else:
                kind = "final"
            step(ch0 + t, t % 6, kind)
        plsc.subcore_barrier()

        sega = rows_a.at[: _CS]
        segb = rows_b.at[: _CS]

        def out_copy(k, carry):
            q0 = r0 + 2 * k * _CS
            q1 = q0 + _CS
            pltpu.sync_copy(acc.at[pl.ds(q0, _CS)], sega)
            pltpu.async_copy(sega, out_hbm.at[c, pl.ds(q0, _CS)], gs0)
            pltpu.sync_copy(acc.at[pl.ds(q1, _CS)], segb)
            pltpu.async_copy(segb, out_hbm.at[c, pl.ds(q1, _CS)], gs1)
            pltpu.make_async_copy(sega, out_hbm.at[c, pl.ds(q0, _CS)], gs0).wait()
            pltpu.make_async_copy(segb, out_hbm.at[c, pl.ds(q1, _CS)], gs1).wait()
            return carry

        lax.fori_loop(0, _KSEG // 2, out_copy, 0)

    return agg_kernel


_agg_x = _make_agg(_D0)
_agg_h = _make_agg(_HP)


def _prep_body(deg_ref, x_ref, xs_ref, nrm_ref):
    d = deg_ref[...]
    onorm = lax.rsqrt(jnp.maximum(d[:, 0:1] + d[:, 2:3], 1.0))
    inorm = lax.rsqrt(jnp.maximum(d[:, 1:2] + d[:, 3:4], 1.0))
    xs_ref[...] = x_ref[...] * onorm
    nrm_ref[...] = jnp.concatenate([onorm, inorm], axis=1)


def _dense1_body(a0_ref, a1_ref, nrm_ref, w_ref, b_ref, out_ref):
    n = nrm_ref[...]
    agg = (a0_ref[...] + a1_ref[...]) * n[:, 1:2]
    h = jnp.tanh(jnp.dot(agg, w_ref[...], preferred_element_type=jnp.float32)
                 + b_ref[...])
    out_ref[...] = h * n[:, 0:1]


def _dense2_body(a0_ref, a1_ref, nrm_ref, x_ref, w2_ref, b2_ref, q1h_ref,
                 q1x_ref, q1b_ref, q2_ref, q2b_ref, q3_ref, q3b_ref, out_ref):
    n = nrm_ref[...]
    agg = (a0_ref[...] + a1_ref[...]) * n[:, 1:2]
    h2 = jnp.tanh(jnp.dot(agg, w2_ref[...], preferred_element_type=jnp.float32)
                  + b2_ref[...])
    y = jnp.tanh(jnp.dot(h2, q1h_ref[...], preferred_element_type=jnp.float32)
                 + jnp.dot(x_ref[...], q1x_ref[...],
                           preferred_element_type=jnp.float32)
                 + q1b_ref[...])
    y = jnp.tanh(jnp.dot(y, q2_ref[...], preferred_element_type=jnp.float32)
                 + q2b_ref[...])
    y = jnp.tanh(jnp.dot(y, q3_ref[...], preferred_element_type=jnp.float32)
                 + q3b_ref[...])
    out_ref[...] = y


def _row_spec(cols):
    return pl.BlockSpec((_R, cols), lambda i: (i, 0))


def _full_spec(shape):
    nd = len(shape)
    return pl.BlockSpec(shape, lambda i, _n=nd: (0,) * _n)


def kernel(x, edge_index, W1, b1, W2, b2, pW1, pb1, pW2, pb2, pW3, pb3):
    ei = edge_index.astype(jnp.int32)
    idx2 = jnp.stack([ei[0].reshape(_E // _C, _C),
                      ei[1].reshape(_E // _C, _C)], axis=1)

    degs = _deg_kernel(idx2).reshape(_NC, 2, _NP)[:, :, :_N]
    degs_t = jnp.transpose(degs, (2, 0, 1)).reshape(_N, 4)

    xs, norms = pl.pallas_call(
        _prep_body,
        grid=(_G,),
        in_specs=[_row_spec(4), _row_spec(_D0)],
        out_specs=[_row_spec(_D0), _row_spec(2)],
        out_shape=[
            jax.ShapeDtypeStruct((_N, _D0), jnp.float32),
            jax.ShapeDtypeStruct((_N, 2), jnp.float32),
        ],
    )(degs_t, x)

    agg1 = _agg_x(xs, idx2)

    w1p = jnp.zeros((_D0, _HP), jnp.float32).at[:, :_H].set(W1)
    b1p = jnp.zeros((1, _HP), jnp.float32).at[0, :_H].set(b1)
    h1s = pl.pallas_call(
        _dense1_body,
        grid=(_G,),
        in_specs=[_row_spec(_D0), _row_spec(_D0), _row_spec(2),
                  _full_spec((_D0, _HP)), _full_spec((1, _HP))],
        out_specs=_row_spec(_HP),
        out_shape=jax.ShapeDtypeStruct((_N, _HP), jnp.float32),
    )(agg1[0], agg1[1], norms, w1p, b1p)

    agg2 = _agg_h(h1s, idx2)

    w2p = jnp.zeros((_HP, _H), jnp.float32).at[:_H, :].set(W2)
    q1h = pW1[:_H]
    q1x = pW1[_H:]
    y = pl.pallas_call(
        _dense2_body,
        grid=(_G,),
        in_specs=[_row_spec(_HP), _row_spec(_HP), _row_spec(2), _row_spec(_D0),
                  _full_spec((_HP, _H)), _full_spec((1, _H)),
                  _full_spec((_H, _H)), _full_spec((_D0, _H)),
                  _full_spec((1, _H)), _full_spec((_H, _H)),
                  _full_spec((1, _H)), _full_spec((_H, 1)),
                  _full_spec((1, 1))],
        out_specs=_row_spec(1),
        out_shape=jax.ShapeDtypeStruct((_N, 1), jnp.float32),
    )(agg2[0], agg2[1], norms, x, w2p, b2.reshape(1, _H), q1h, q1x,
      pb1.reshape(1, _H), pW2, pb2.reshape(1, _H), pW3, pb3.reshape(1, 1))
    return y


# speedup vs baseline: 1.0432x; 1.0432x over previous
import functools

import jax
import jax.numpy as jnp
from jax import lax
from jax.experimental import pallas as pl
from jax.experimental.pallas import tpu as pltpu
from jax.experimental.pallas import tpu_sc as plsc

_N = 10000
_E = 320000
_D0 = 128
_H = 150
_HP = 160
_NC = 2
_NS = 16
_NW = _NC * _NS
_EPT = _E // _NW
_C = 80
_NCH = _EPT // _C
_RPTP = 640
_NP = _RPTP * _NS
_CS = 80
_KSEG = _RPTP // _CS
_R = 1000
_G = _N // _R


def _sc_mesh():
    return plsc.VectorSubcoreMesh(core_axis_name="c", subcore_axis_name="s")


@functools.partial(
    pl.kernel,
    mesh=_sc_mesh(),
    out_type=jax.ShapeDtypeStruct((_NC * 2 * _NP,), jnp.float32),
    compiler_params=pltpu.CompilerParams(use_tc_tiling_on_sc=False),
    scratch_types=[
        pltpu.VMEM((_NCH, 2, _C), jnp.int32),
        pltpu.VMEM((_C,), jnp.float32),
        pltpu.VMEM((_RPTP,), jnp.float32),
        pltpu.VMEM_SHARED((_NP,), jnp.float32),
        pltpu.VMEM_SHARED((_NP,), jnp.float32),
        pltpu.SemaphoreType.DMA,
        pltpu.SemaphoreType.DMA,
    ],
)
def _deg_kernel(idx_hbm, out_hbm, ixall, ones, stage, degs, degd, ds0, ds1):
    c = lax.axis_index("c")
    s = lax.axis_index("s")
    ch0 = (s * _NC + c) * _NCH
    pltpu.sync_copy(idx_hbm.at[pl.ds(ch0, _NCH)], ixall)
    for j in range(_C // 16):
        ones[pl.ds(j * 16, 16)] = jnp.ones((16,), jnp.float32)

    def zfill(j, carry):
        stage[pl.ds(j * 16, 16)] = jnp.zeros((16,), jnp.float32)
        return carry

    lax.fori_loop(0, _RPTP // 16, zfill, 0)
    r0 = s * _RPTP
    pltpu.sync_copy(stage, degs.at[pl.ds(r0, _RPTP)])
    pltpu.sync_copy(stage, degd.at[pl.ds(r0, _RPTP)])
    plsc.subcore_barrier()
    one_c = ones

    def body(i, carry):
        pltpu.sync_copy(one_c, degs.at[ixall.at[i, 0]], add=True)
        pltpu.sync_copy(one_c, degd.at[ixall.at[i, 1]], add=True)
        return carry

    lax.fori_loop(0, _NCH, body, 0)
    plsc.subcore_barrier()
    pltpu.sync_copy(degs.at[pl.ds(r0, _RPTP)], stage)
    pltpu.sync_copy(stage, out_hbm.at[pl.ds(c * 2 * _NP + r0, _RPTP)])
    pltpu.sync_copy(degd.at[pl.ds(r0, _RPTP)], stage)
    pltpu.sync_copy(stage, out_hbm.at[pl.ds((c * 2 + 1) * _NP + r0, _RPTP)])


def _make_agg(D):
    @functools.partial(
        pl.kernel,
        mesh=_sc_mesh(),
        out_type=jax.ShapeDtypeStruct((_NC, _NP, D), jnp.float32),
        compiler_params=pltpu.CompilerParams(use_tc_tiling_on_sc=False),
        scratch_types=[
            pltpu.VMEM((2, _C), jnp.int32),
            pltpu.VMEM((2, _C), jnp.int32),
            pltpu.VMEM((_C, D), jnp.float32),
            pltpu.VMEM((_C, D), jnp.float32),
            pltpu.VMEM_SHARED((_NP, D), jnp.float32),
            pltpu.SemaphoreType.DMA,
            pltpu.SemaphoreType.DMA,
            pltpu.SemaphoreType.DMA,
            pltpu.SemaphoreType.DMA,
        ],
    )
    def agg_kernel(h_hbm, idx_hbm, out_hbm, ixa, ixb, rows_a, rows_b,
                   acc, gs0, gs1, isa, isb):
        c = lax.axis_index("c")
        s = lax.axis_index("s")
        r0 = s * _RPTP
        ch0 = (s * _NC + c) * _NCH

        def zfill(r, carry):
            for j in range(D // 16):
                rows_a[r, pl.ds(j * 16, 16)] = jnp.zeros((16,), jnp.float32)
            return carry

        lax.fori_loop(0, _CS, zfill, 0)
        seg = rows_a.at[: _CS]

        def zcopy(k, carry):
            pltpu.sync_copy(seg, acc.at[pl.ds(r0 + k * _CS, _CS)])
            return carry

        lax.fori_loop(0, _KSEG, zcopy, 0)
        plsc.subcore_barrier()

        pltpu.sync_copy(idx_hbm.at[ch0], ixa)
        pltpu.async_copy(h_hbm.at[ixa.at[0]], rows_a, gs0)
        pltpu.async_copy(idx_hbm.at[ch0 + 1], ixb, isb)

        def body(j, carry):
            i0 = ch0 + 2 * j
            i3 = jnp.minimum(i0 + 3, ch0 + _NCH - 1)
            pltpu.make_async_copy(idx_hbm.at[i0 + 1], ixb, isb).wait()
            pltpu.async_copy(h_hbm.at[ixb.at[0]], rows_b, gs1)
            pltpu.make_async_copy(h_hbm.at[ixa.at[0]], rows_a, gs0).wait()
            pltpu.sync_copy(rows_a, acc.at[ixa.at[1]], add=True)
            pltpu.async_copy(idx_hbm.at[i0 + 2], ixa, isa)
            pltpu.make_async_copy(idx_hbm.at[i0 + 2], ixa, isa).wait()
            pltpu.async_copy(h_hbm.at[ixa.at[0]], rows_a, gs0)
            pltpu.make_async_copy(h_hbm.at[ixb.at[0]], rows_b, gs1).wait()
            pltpu.sync_copy(rows_b, acc.at[ixb.at[1]], add=True)
            pltpu.async_copy(idx_hbm.at[i3], ixb, isb)
            return carry

        lax.fori_loop(0, (_NCH - 1) // 2, body, 0)
        pltpu.make_async_copy(h_hbm.at[ixa.at[0]], rows_a, gs0).wait()
        pltpu.sync_copy(rows_a, acc.at[ixa.at[1]], add=True)
        pltpu.make_async_copy(idx_hbm.at[ch0 + _NCH - 1], ixb, isb).wait()
        plsc.subcore_barrier()

        sega = rows_a.at[: _CS]
        segb = rows_b.at[: _CS]

        def out_copy(k, carry):
            q0 = r0 + 2 * k * _CS
            q1 = q0 + _CS
            pltpu.sync_copy(acc.at[pl.ds(q0, _CS)], sega)
            pltpu.async_copy(sega, out_hbm.at[c, pl.ds(q0, _CS)], gs0)
            pltpu.sync_copy(acc.at[pl.ds(q1, _CS)], segb)
            pltpu.async_copy(segb, out_hbm.at[c, pl.ds(q1, _CS)], gs1)
            pltpu.make_async_copy(sega, out_hbm.at[c, pl.ds(q0, _CS)], gs0).wait()
            pltpu.make_async_copy(segb, out_hbm.at[c, pl.ds(q1, _CS)], gs1).wait()
            return carry

        lax.fori_loop(0, _KSEG // 2, out_copy, 0)

    return agg_kernel


_agg_x = _make_agg(_D0)
_agg_h = _make_agg(_HP)


def _prep_body(deg_ref, x_ref, xs_ref, nrm_ref):
    d = deg_ref[...]
    onorm = lax.rsqrt(jnp.maximum(d[:, 0:1] + d[:, 2:3], 1.0))
    inorm = lax.rsqrt(jnp.maximum(d[:, 1:2] + d[:, 3:4], 1.0))
    xs_ref[...] = x_ref[...] * onorm
    nrm_ref[...] = jnp.concatenate([onorm, inorm], axis=1)


def _dense1_body(a0_ref, a1_ref, nrm_ref, w_ref, b_ref, out_ref):
    n = nrm_ref[...]
    agg = (a0_ref[...] + a1_ref[...]) * n[:, 1:2]
    h = jnp.tanh(jnp.dot(agg, w_ref[...], preferred_element_type=jnp.float32)
                 + b_ref[...])
    out_ref[...] = h * n[:, 0:1]


def _dense2_body(a0_ref, a1_ref, nrm_ref, x_ref, w2_ref, b2_ref, q1h_ref,
                 q1x_ref, q1b_ref, q2_ref, q2b_ref, q3_ref, q3b_ref, out_ref):
    n = nrm_ref[...]
    agg = (a0_ref[...] + a1_ref[...]) * n[:, 1:2]
    h2 = jnp.tanh(jnp.dot(agg, w2_ref[...], preferred_element_type=jnp.float32)
                  + b2_ref[...])
    y = jnp.tanh(jnp.dot(h2, q1h_ref[...], preferred_element_type=jnp.float32)
                 + jnp.dot(x_ref[...], q1x_ref[...],
                           preferred_element_type=jnp.float32)
                 + q1b_ref[...])
    y = jnp.tanh(jnp.dot(y, q2_ref[...], preferred_element_type=jnp.float32)
                 + q2b_ref[...])
    y = jnp.tanh(jnp.dot(y, q3_ref[...], preferred_element_type=jnp.float32)
                 + q3b_ref[...])
    out_ref[...] = y


def _row_spec(cols):
    return pl.BlockSpec((_R, cols), lambda i: (i, 0))


def _full_spec(shape):
    nd = len(shape)
    return pl.BlockSpec(shape, lambda i, _n=nd: (0,) * _n)


def kernel(x, edge_index, W1, b1, W2, b2, pW1, pb1, pW2, pb2, pW3, pb3):
    ei = edge_index.astype(jnp.int32)
    idx2 = jnp.stack([ei[0].reshape(_E // _C, _C),
                      ei[1].reshape(_E // _C, _C)], axis=1)

    degs = _deg_kernel(idx2).reshape(_NC, 2, _NP)[:, :, :_N]
    degs_t = jnp.transpose(degs, (2, 0, 1)).reshape(_N, 4)

    xs, norms = pl.pallas_call(
        _prep_body,
        grid=(_G,),
        in_specs=[_row_spec(4), _row_spec(_D0)],
        out_specs=[_row_spec(_D0), _row_spec(2)],
        out_shape=[
            jax.ShapeDtypeStruct((_N, _D0), jnp.float32),
            jax.ShapeDtypeStruct((_N, 2), jnp.float32),
        ],
    )(degs_t, x)

    agg1 = _agg_x(xs, idx2)

    w1p = jnp.zeros((_D0, _HP), jnp.float32).at[:, :_H].set(W1)
    b1p = jnp.zeros((1, _HP), jnp.float32).at[0, :_H].set(b1)
    h1s = pl.pallas_call(
        _dense1_body,
        grid=(_G,),
        in_specs=[_row_spec(_D0), _row_spec(_D0), _row_spec(2),
                  _full_spec((_D0, _HP)), _full_spec((1, _HP))],
        out_specs=_row_spec(_HP),
        out_shape=jax.ShapeDtypeStruct((_N, _HP), jnp.float32),
    )(agg1[0], agg1[1], norms, w1p, b1p)

    agg2 = _agg_h(h1s, idx2)

    w2p = jnp.zeros((_HP, _H), jnp.float32).at[:_H, :].set(W2)
    q1h = pW1[:_H]
    q1x = pW1[_H:]
    y = pl.pallas_call(
        _dense2_body,
        grid=(_G,),
        in_specs=[_row_spec(_HP), _row_spec(_HP), _row_spec(2), _row_spec(_D0),
                  _full_spec((_HP, _H)), _full_spec((1, _H)),
                  _full_spec((_H, _H)), _full_spec((_D0, _H)),
                  _full_spec((1, _H)), _full_spec((_H, _H)),
                  _full_spec((1, _H)), _full_spec((_H, 1)),
                  _full_spec((1, 1))],
        out_specs=_row_spec(1),
        out_shape=jax.ShapeDtypeStruct((_N, 1), jnp.float32),
    )(agg2[0], agg2[1], norms, x, w2p, b2.reshape(1, _H), q1h, q1x,
      pb1.reshape(1, _H), pW2, pb2.reshape(1, _H), pW3, pb3.reshape(1, 1))
    return y


# speedup vs baseline: 1.0771x; 1.0325x over previous
import functools

import jax
import jax.numpy as jnp
from jax import lax
from jax.experimental import pallas as pl
from jax.experimental.pallas import tpu as pltpu
from jax.experimental.pallas import tpu_sc as plsc

_N = 10000
_E = 320000
_D0 = 128
_H = 150
_HP = 160
_NC = 2
_NS = 16
_NW = _NC * _NS
_EPT = _E // _NW
_C = 80
_NCH = _EPT // _C
_RPTP = 640
_NP = _RPTP * _NS
_CS = 80
_KSEG = _RPTP // _CS
_R = 1000
_G = _N // _R


def _sc_mesh():
    return plsc.VectorSubcoreMesh(core_axis_name="c", subcore_axis_name="s")


@functools.partial(
    pl.kernel,
    mesh=_sc_mesh(),
    out_type=jax.ShapeDtypeStruct((_NC * 2 * _NP,), jnp.float32),
    compiler_params=pltpu.CompilerParams(use_tc_tiling_on_sc=False),
    scratch_types=[
        pltpu.VMEM((_NCH, 2, _C), jnp.int32),
        pltpu.VMEM((_C,), jnp.float32),
        pltpu.VMEM((_RPTP,), jnp.float32),
        pltpu.VMEM_SHARED((_NP,), jnp.float32),
        pltpu.VMEM_SHARED((_NP,), jnp.float32),
        pltpu.SemaphoreType.DMA,
        pltpu.SemaphoreType.DMA,
    ],
)
def _deg_kernel(idx_hbm, out_hbm, ixall, ones, stage, degs, degd, ds0, ds1):
    c = lax.axis_index("c")
    s = lax.axis_index("s")
    ch0 = (s * _NC + c) * _NCH
    pltpu.sync_copy(idx_hbm.at[pl.ds(ch0, _NCH)], ixall)
    for j in range(_C // 16):
        ones[pl.ds(j * 16, 16)] = jnp.ones((16,), jnp.float32)

    def zfill(j, carry):
        stage[pl.ds(j * 16, 16)] = jnp.zeros((16,), jnp.float32)
        return carry

    lax.fori_loop(0, _RPTP // 16, zfill, 0)
    r0 = s * _RPTP
    pltpu.sync_copy(stage, degs.at[pl.ds(r0, _RPTP)])
    pltpu.sync_copy(stage, degd.at[pl.ds(r0, _RPTP)])
    plsc.subcore_barrier()
    one_c = ones

    def body(m, carry):
        for t in range(5):
            i = m * 5 + t
            pltpu.async_copy(one_c, degs.at[ixall.at[i, 0]], ds0, add=True)
            pltpu.async_copy(one_c, degd.at[ixall.at[i, 1]], ds1, add=True)
        for t in range(5):
            i = m * 5 + t
            pltpu.make_async_copy(one_c, degs.at[ixall.at[i, 0]], ds0).wait()
            pltpu.make_async_copy(one_c, degd.at[ixall.at[i, 1]], ds1).wait()
        return carry

    lax.fori_loop(0, _NCH // 5, body, 0)
    plsc.subcore_barrier()
    pltpu.sync_copy(degs.at[pl.ds(r0, _RPTP)], stage)
    pltpu.sync_copy(stage, out_hbm.at[pl.ds(c * 2 * _NP + r0, _RPTP)])
    pltpu.sync_copy(degd.at[pl.ds(r0, _RPTP)], stage)
    pltpu.sync_copy(stage, out_hbm.at[pl.ds((c * 2 + 1) * _NP + r0, _RPTP)])


def _make_agg(D):
    @functools.partial(
        pl.kernel,
        mesh=_sc_mesh(),
        out_type=jax.ShapeDtypeStruct((_NC, _NP, D), jnp.float32),
        compiler_params=pltpu.CompilerParams(use_tc_tiling_on_sc=False),
        scratch_types=[
            pltpu.VMEM((2, _C), jnp.int32),
            pltpu.VMEM((2, _C), jnp.int32),
            pltpu.VMEM((_C, D), jnp.float32),
            pltpu.VMEM((_C, D), jnp.float32),
            pltpu.VMEM_SHARED((_NP, D), jnp.float32),
            pltpu.SemaphoreType.DMA,
            pltpu.SemaphoreType.DMA,
            pltpu.SemaphoreType.DMA,
            pltpu.SemaphoreType.DMA,
        ],
    )
    def agg_kernel(h_hbm, idx_hbm, out_hbm, ixa, ixb, rows_a, rows_b,
                   acc, gs0, gs1, isa, isb):
        c = lax.axis_index("c")
        s = lax.axis_index("s")
        r0 = s * _RPTP
        ch0 = (s * _NC + c) * _NCH

        def zfill(r, carry):
            for j in range(D // 16):
                rows_a[r, pl.ds(j * 16, 16)] = jnp.zeros((16,), jnp.float32)
            return carry

        lax.fori_loop(0, _CS, zfill, 0)
        seg = rows_a.at[: _CS]

        def zcopy(k, carry):
            pltpu.sync_copy(seg, acc.at[pl.ds(r0 + k * _CS, _CS)])
            return carry

        lax.fori_loop(0, _KSEG, zcopy, 0)
        plsc.subcore_barrier()

        pltpu.sync_copy(idx_hbm.at[ch0], ixa)
        pltpu.async_copy(h_hbm.at[ixa.at[0]], rows_a, gs0)
        pltpu.async_copy(idx_hbm.at[ch0 + 1], ixb, isb)

        def body(j, carry):
            i0 = ch0 + 2 * j
            i3 = jnp.minimum(i0 + 3, ch0 + _NCH - 1)
            pltpu.make_async_copy(idx_hbm.at[i0 + 1], ixb, isb).wait()
            pltpu.async_copy(h_hbm.at[ixb.at[0]], rows_b, gs1)
            pltpu.make_async_copy(h_hbm.at[ixa.at[0]], rows_a, gs0).wait()
            pltpu.sync_copy(rows_a, acc.at[ixa.at[1]], add=True)
            pltpu.async_copy(idx_hbm.at[i0 + 2], ixa, isa)
            pltpu.make_async_copy(idx_hbm.at[i0 + 2], ixa, isa).wait()
            pltpu.async_copy(h_hbm.at[ixa.at[0]], rows_a, gs0)
            pltpu.make_async_copy(h_hbm.at[ixb.at[0]], rows_b, gs1).wait()
            pltpu.sync_copy(rows_b, acc.at[ixb.at[1]], add=True)
            pltpu.async_copy(idx_hbm.at[i3], ixb, isb)
            return carry

        lax.fori_loop(0, (_NCH - 1) // 2, body, 0)
        pltpu.make_async_copy(h_hbm.at[ixa.at[0]], rows_a, gs0).wait()
        pltpu.sync_copy(rows_a, acc.at[ixa.at[1]], add=True)
        pltpu.make_async_copy(idx_hbm.at[ch0 + _NCH - 1], ixb, isb).wait()
        plsc.subcore_barrier()

        sega = rows_a.at[: _CS]
        segb = rows_b.at[: _CS]

        def out_copy(k, carry):
            q0 = r0 + 2 * k * _CS
            q1 = q0 + _CS
            pltpu.sync_copy(acc.at[pl.ds(q0, _CS)], sega)
            pltpu.async_copy(sega, out_hbm.at[c, pl.ds(q0, _CS)], gs0)
            pltpu.sync_copy(acc.at[pl.ds(q1, _CS)], segb)
            pltpu.async_copy(segb, out_hbm.at[c, pl.ds(q1, _CS)], gs1)
            pltpu.make_async_copy(sega, out_hbm.at[c, pl.ds(q0, _CS)], gs0).wait()
            pltpu.make_async_copy(segb, out_hbm.at[c, pl.ds(q1, _CS)], gs1).wait()
            return carry

        lax.fori_loop(0, _KSEG // 2, out_copy, 0)

    return agg_kernel


_agg_x = _make_agg(_D0)
_agg_h = _make_agg(_HP)


def _prep_body(deg_ref, x_ref, xs_ref, nrm_ref):
    d = deg_ref[...]
    onorm = lax.rsqrt(jnp.maximum(d[:, 0:1] + d[:, 2:3], 1.0))
    inorm = lax.rsqrt(jnp.maximum(d[:, 1:2] + d[:, 3:4], 1.0))
    xs_ref[...] = x_ref[...] * onorm
    nrm_ref[...] = jnp.concatenate([onorm, inorm], axis=1)


def _dense1_body(a0_ref, a1_ref, nrm_ref, w_ref, b_ref, out_ref):
    n = nrm_ref[...]
    agg = (a0_ref[...] + a1_ref[...]) * n[:, 1:2]
    h = jnp.tanh(jnp.dot(agg, w_ref[...], preferred_element_type=jnp.float32)
                 + b_ref[...])
    out_ref[...] = h * n[:, 0:1]


def _dense2_body(a0_ref, a1_ref, nrm_ref, x_ref, w2_ref, b2_ref, q1h_ref,
                 q1x_ref, q1b_ref, q2_ref, q2b_ref, q3_ref, q3b_ref, out_ref):
    n = nrm_ref[...]
    agg = (a0_ref[...] + a1_ref[...]) * n[:, 1:2]
    h2 = jnp.tanh(jnp.dot(agg, w2_ref[...], preferred_element_type=jnp.float32)
                  + b2_ref[...])
    y = jnp.tanh(jnp.dot(h2, q1h_ref[...], preferred_element_type=jnp.float32)
                 + jnp.dot(x_ref[...], q1x_ref[...],
                           preferred_element_type=jnp.float32)
                 + q1b_ref[...])
    y = jnp.tanh(jnp.dot(y, q2_ref[...], preferred_element_type=jnp.float32)
                 + q2b_ref[...])
    y = jnp.tanh(jnp.dot(y, q3_ref[...], preferred_element_type=jnp.float32)
                 + q3b_ref[...])
    out_ref[...] = y


def _row_spec(cols):
    return pl.BlockSpec((_R, cols), lambda i: (i, 0))


def _full_spec(shape):
    nd = len(shape)
    return pl.BlockSpec(shape, lambda i, _n=nd: (0,) * _n)


def kernel(x, edge_index, W1, b1, W2, b2, pW1, pb1, pW2, pb2, pW3, pb3):
    ei = edge_index.astype(jnp.int32)
    idx2 = jnp.stack([ei[0].reshape(_E // _C, _C),
                      ei[1].reshape(_E // _C, _C)], axis=1)

    degs = _deg_kernel(idx2).reshape(_NC, 2, _NP)[:, :, :_N]
    degs_t = jnp.transpose(degs, (2, 0, 1)).reshape(_N, 4)

    xs, norms = pl.pallas_call(
        _prep_body,
        grid=(_G,),
        in_specs=[_row_spec(4), _row_spec(_D0)],
        out_specs=[_row_spec(_D0), _row_spec(2)],
        out_shape=[
            jax.ShapeDtypeStruct((_N, _D0), jnp.float32),
            jax.ShapeDtypeStruct((_N, 2), jnp.float32),
        ],
    )(degs_t, x)

    agg1 = _agg_x(xs, idx2)

    w1p = jnp.zeros((_D0, _HP), jnp.float32).at[:, :_H].set(W1)
    b1p = jnp.zeros((1, _HP), jnp.float32).at[0, :_H].set(b1)
    h1s = pl.pallas_call(
        _dense1_body,
        grid=(_G,),
        in_specs=[_row_spec(_D0), _row_spec(_D0), _row_spec(2),
                  _full_spec((_D0, _HP)), _full_spec((1, _HP))],
        out_specs=_row_spec(_HP),
        out_shape=jax.ShapeDtypeStruct((_N, _HP), jnp.float32),
    )(agg1[0], agg1[1], norms, w1p, b1p)

    agg2 = _agg_h(h1s, idx2)

    w2p = jnp.zeros((_HP, _H), jnp.float32).at[:_H, :].set(W2)
    q1h = pW1[:_H]
    q1x = pW1[_H:]
    y = pl.pallas_call(
        _dense2_body,
        grid=(_G,),
        in_specs=[_row_spec(_HP), _row_spec(_HP), _row_spec(2), _row_spec(_D0),
                  _full_spec((_HP, _H)), _full_spec((1, _H)),
                  _full_spec((_H, _H)), _full_spec((_D0, _H)),
                  _full_spec((1, _H)), _full_spec((_H, _H)),
                  _full_spec((1, _H)), _full_spec((_H, 1)),
                  _full_spec((1, 1))],
        out_specs=_row_spec(1),
        out_shape=jax.ShapeDtypeStruct((_N, 1), jnp.float32),
    )(agg2[0], agg2[1], norms, x, w2p, b2.reshape(1, _H), q1h, q1x,
      pb1.reshape(1, _H), pW2, pb2.reshape(1, _H), pW3, pb3.reshape(1, 1))
    return y
